# baseline (speedup 1.0000x reference)
import functools
import math

import jax
import jax.numpy as jnp
from jax import lax
from jax.experimental import pallas as pl
from jax.experimental.pallas import tpu as pltpu
from jax.experimental.pallas import tpu_sc as plsc

D_MODEL = 64
SCALE = math.sqrt(D_MODEL)

NC = 2
NS = 16
NW = NC * NS
LANES = 16
NBUF = 4

VOCAB_BLK = 128


@jax.jit
def _prep_table(tab_t, tail2):
    vocab = tab_t.shape[1]
    n_blk = vocab // VOCAB_BLK
    n_pair = vocab // 2
    mesh = plsc.VectorSubcoreMesh(core_axis_name="c", subcore_axis_name="s")

    @functools.partial(
        pl.kernel,
        mesh=mesh,
        out_type=jax.ShapeDtypeStruct((n_pair, 2 * D_MODEL), jnp.float32),
        scratch_types=[
            pltpu.VMEM((D_MODEL, VOCAB_BLK), jnp.float32),
            pltpu.VMEM((VOCAB_BLK // 2, 2 * D_MODEL), jnp.float32),
            pltpu.SemaphoreType.DMA,
        ],
        compiler_params=pltpu.CompilerParams(needs_layout_passes=False),
    )
    def p1(tab_hbm, tail_hbm, out_hbm, inb, outb, sem):
        wid = lax.axis_index("s") * NC + lax.axis_index("c")

        rowvs = [
            lax.iota(jnp.int32, LANES) + (qg * LANES) % D_MODEL
            for qg in range(2 * D_MODEL // LANES)
        ]

        @pl.loop(wid, n_blk, step=NW)
        def _blk(blk):
            c0 = pl.multiple_of(blk * VOCAB_BLK, VOCAB_BLK)
            pltpu.sync_copy(tab_hbm.at[:, pl.ds(c0, VOCAB_BLK)], inb)

            @pl.loop(0, VOCAB_BLK // 2)
            def _row(p):
                for qg in range(2 * D_MODEL // LANES):
                    colv = rowvs[qg] * 0 + (2 * p + qg // (D_MODEL // LANES))
                    vals = plsc.load_gather(inb, [rowvs[qg], colv])
                    outb[p, pl.ds(qg * LANES, LANES)] = vals * SCALE

            pltpu.sync_copy(outb, out_hbm.at[pl.ds(blk * (VOCAB_BLK // 2),
                                                   VOCAB_BLK // 2)])

        @pl.when(wid == 0)
        def _tail():
            pltpu.sync_copy(tail_hbm, outb.at[pl.ds(0, tail2.shape[0])])
            pltpu.sync_copy(outb.at[pl.ds(0, tail2.shape[0])],
                            out_hbm.at[pl.ds(n_blk * (VOCAB_BLK // 2),
                                             tail2.shape[0])])

    return p1(tab_t, tail2)


@functools.partial(jax.jit, static_argnames=("n_seq", "seq_len"))
def _lookup(idx_flat, tab_lin, *, n_seq, seq_len):
    seq_per_w = n_seq // NW
    n_idx_w = seq_per_w * seq_len
    mesh = plsc.VectorSubcoreMesh(core_axis_name="c", subcore_axis_name="s")

    @functools.partial(
        pl.kernel,
        mesh=mesh,
        out_type=jax.ShapeDtypeStruct((n_seq, seq_len, D_MODEL), jnp.float32),
        scratch_types=[
            pltpu.VMEM((n_idx_w,), jnp.int32),
            pltpu.VMEM((NBUF, seq_len, D_MODEL), jnp.float32),
            pltpu.SemaphoreType.DMA,
            *([pltpu.SemaphoreType.DMA] * NBUF),
            *([pltpu.SemaphoreType.DMA] * NBUF),
        ],
        compiler_params=pltpu.CompilerParams(use_tc_tiling_on_sc=False),
    )
    def k(idx_hbm, tab_hbm, out_hbm, idx_v, rows_v, isem, gs0, gs1, gs2,
          gs3, ss0, ss1, ss2, ss3):
        gsem = (gs0, gs1, gs2, gs3)
        ssem = (ss0, ss1, ss2, ss3)
        wid = lax.axis_index("s") * NC + lax.axis_index("c")
        seq0 = wid * seq_per_w

        pltpu.async_copy(
            idx_hbm.at[pl.ds(seq0 * seq_len, n_idx_w)], idx_v, isem).wait()

        def gather(c, b):
            return pltpu.make_async_copy(
                tab_hbm.at[idx_v.at[pl.ds(c * seq_len, seq_len)]],
                rows_v.at[b], gsem[b])

        def scatter(c, b):
            return pltpu.make_async_copy(
                rows_v.at[b], out_hbm.at[seq0 + c], ssem[b])

        gather(0, 0).start()
        gather(1, 1).start()

        @pl.loop(0, seq_per_w, step=NBUF)
        def _outer(t):
            for b in range(NBUF):
                c = t + b
                f = (b + 2) % NBUF
                cn = c + 2

                gather(c, b).wait()

                @pl.when(cn < seq_per_w)
                def _prefetch():
                    @pl.when(cn >= NBUF)
                    def _drain():
                        scatter(cn - NBUF, f).wait()

                    gather(cn, f).start()

                scatter(c, b).start()

        for b in range(NBUF):
            scatter(seq_per_w - NBUF + b, b).wait()

    return k(idx_flat, tab_lin)


def kernel(input_ids, table):
    n_seq, seq_len = input_ids.shape
    vocab = table.shape[0]
    idx_flat = input_ids.reshape(-1).astype(jnp.int32)

    v_main = vocab // VOCAB_BLK * VOCAB_BLK
    tail2 = (table[v_main:] * SCALE).reshape(-1, 2 * D_MODEL)
    tab_c = _prep_table(table.T, tail2)
    tab_lin = tab_c.reshape(vocab, D_MODEL)
    return _lookup(idx_flat, tab_lin, n_seq=n_seq, seq_len=seq_len)


# speedup vs baseline: 2.3515x; 2.3515x over previous
import functools
import math

import jax
import jax.numpy as jnp
from jax import lax
from jax.experimental import pallas as pl
from jax.experimental.pallas import tpu as pltpu
from jax.experimental.pallas import tpu_sc as plsc

D_MODEL = 64
SCALE = math.sqrt(D_MODEL)

NC = 2
NS = 16
NW = NC * NS
LANES = 16
NBUF = 4


@functools.partial(jax.jit, static_argnames=("n_seq", "seq_len"))
def _embed(idx_flat, tab_pad, *, n_seq, seq_len):
    seq_per_w = n_seq // NW
    n_idx_w = seq_per_w * seq_len
    mesh = plsc.VectorSubcoreMesh(core_axis_name="c", subcore_axis_name="s")

    @functools.partial(
        pl.kernel,
        mesh=mesh,
        out_type=jax.ShapeDtypeStruct((n_seq, seq_len, D_MODEL), jnp.float32),
        scratch_types=[
            pltpu.VMEM((n_idx_w,), jnp.int32),
            pltpu.VMEM((2, seq_len, 2 * D_MODEL), jnp.float32),
            pltpu.VMEM((2, seq_len, D_MODEL), jnp.float32),
            pltpu.SemaphoreType.DMA,
            *([pltpu.SemaphoreType.DMA] * 2),
            *([pltpu.SemaphoreType.DMA] * 2),
        ],
    )
    def k(idx_hbm, tab_hbm, out_hbm, idx_v, in_v, out_v, isem, gs0, gs1,
          ss0, ss1):
        gsem = (gs0, gs1)
        ssem = (ss0, ss1)
        wid = lax.axis_index("s") * NC + lax.axis_index("c")
        seq0 = wid * seq_per_w

        pltpu.async_copy(
            idx_hbm.at[pl.ds(seq0 * seq_len, n_idx_w)], idx_v, isem).wait()

        def gather(c, b):
            return pltpu.make_async_copy(
                tab_hbm.at[idx_v.at[pl.ds(c * seq_len, seq_len)]],
                in_v.at[b], gsem[b])

        def scatter(c, b):
            return pltpu.make_async_copy(
                out_v.at[b], out_hbm.at[seq0 + c], ssem[b])

        gather(0, 0).start()
        gather(1, 1).start()

        @pl.loop(0, seq_per_w, step=2)
        def _outer(t):
            for b in range(2):
                c = t + b
                gather(c, b).wait()

                @pl.when(c >= 2)
                def _drain():
                    scatter(c - 2, b).wait()

                src = in_v.at[b]
                dst = out_v.at[b]

                @pl.loop(0, seq_len)
                def _scale(r):
                    for j in range(D_MODEL // LANES):
                        sl = pl.ds(j * LANES, LANES)
                        dst[r, sl] = src[r, sl] * SCALE

                scatter(c, b).start()

                @pl.when(c + 2 < seq_per_w)
                def _prefetch():
                    gather(c + 2, b).start()

        for b in range(2):
            scatter(seq_per_w - 2 + b, b).wait()

    return k(idx_flat, tab_pad)


def kernel(input_ids, table):
    n_seq, seq_len = input_ids.shape
    idx_flat = input_ids.reshape(-1).astype(jnp.int32)
    tab_pad = jnp.pad(table.T, ((0, D_MODEL), (0, 0))).T
    return _embed(idx_flat, tab_pad, n_seq=n_seq, seq_len=seq_len)


# speedup vs baseline: 2.3581x; 1.0028x over previous
import functools
import math

import jax
import jax.numpy as jnp
from jax import lax
from jax.experimental import pallas as pl
from jax.experimental.pallas import tpu as pltpu
from jax.experimental.pallas import tpu_sc as plsc

D_MODEL = 64
SCALE = math.sqrt(D_MODEL)

NC = 2
NS = 16
NW = NC * NS
LANES = 16
NBUF = 4


@functools.partial(jax.jit, static_argnames=("n_seq", "seq_len"))
def _embed(idx_flat, tab_pad, *, n_seq, seq_len):
    seq_per_w = n_seq // NW
    n_idx_w = seq_per_w * seq_len
    mesh = plsc.VectorSubcoreMesh(core_axis_name="c", subcore_axis_name="s")

    @functools.partial(
        pl.kernel,
        mesh=mesh,
        out_type=jax.ShapeDtypeStruct((n_seq, seq_len, D_MODEL), jnp.float32),
        scratch_types=[
            pltpu.VMEM((n_idx_w,), jnp.int32),
            pltpu.VMEM((2, seq_len, 2 * D_MODEL), jnp.float32),
            pltpu.VMEM((2, seq_len, D_MODEL), jnp.float32),
            pltpu.SemaphoreType.DMA,
            *([pltpu.SemaphoreType.DMA] * 2),
            *([pltpu.SemaphoreType.DMA] * 2),
        ],
    )
    def k(idx_hbm, tab_hbm, out_hbm, idx_v, in_v, out_v, isem, gs0, gs1,
          ss0, ss1):
        gsem = (gs0, gs1)
        ssem = (ss0, ss1)
        wid = lax.axis_index("s") * NC + lax.axis_index("c")
        seq0 = wid * seq_per_w

        pltpu.async_copy(
            idx_hbm.at[pl.ds(seq0 * seq_len, n_idx_w)], idx_v, isem).wait()

        def gather(c, b):
            return pltpu.make_async_copy(
                tab_hbm.at[idx_v.at[pl.ds(c * seq_len, seq_len)]],
                in_v.at[b], gsem[b])

        def scatter(c, b):
            return pltpu.make_async_copy(
                out_v.at[b], out_hbm.at[seq0 + c], ssem[b])

        gather(0, 0).start()
        gather(1, 1).start()

        @pl.loop(0, seq_per_w, step=2)
        def _outer(t):
            for b in range(2):
                c = t + b
                gather(c, b).wait()

                @pl.when(c >= 2)
                def _drain():
                    scatter(c - 2, b).wait()

                src = in_v.at[b]
                dst = out_v.at[b]

                @pl.loop(0, seq_len)
                def _scale(r):
                    for j in range(D_MODEL // LANES):
                        sl = pl.ds(j * LANES, LANES)
                        dst[r, sl] = src[r, sl] * SCALE

                scatter(c, b).start()

                @pl.when(c + 2 < seq_per_w)
                def _prefetch():
                    gather(c + 2, b).start()

        for b in range(2):
            scatter(seq_per_w - 2 + b, b).wait()

    return k(idx_flat, tab_pad)


def kernel(input_ids, table):
    n_seq, seq_len = input_ids.shape
    idx_flat = input_ids.reshape(-1).astype(jnp.int32)
    tab_pad = jnp.pad(table, ((0, 0), (0, D_MODEL)))
    return _embed(idx_flat, tab_pad, n_seq=n_seq, seq_len=seq_len)
